# baseline (speedup 1.0000x reference)
import functools

import jax
import jax.numpy as jnp
from jax.experimental import pallas as pl
from jax.experimental.pallas import tpu as pltpu

NV = 16384
D_HID = 128
D_ROUTE = 64
D_IN = 2 * D_HID + D_ROUTE
D_FUSE_HID = 256
D_FUSE_OUT = 192
N_EXPERTS = 8
EH = N_EXPERTS * D_FUSE_OUT
BLK = 2048


_GELU_C = 0.7978845608028654
_GELU_A = 0.044715


def _gelu_fast(x):
    x2 = x * x
    m = x2 * (_GELU_C * _GELU_A) + _GELU_C
    t = jnp.tanh(x * m)
    hx = 0.5 * x
    return hx * t + hx


def _body(veh_ref, ctx_ref, route_ref, w1_ref, b1_ref,
          ln_g_ref, ln_b_ref, w2_ref, b2_ref, gate_w_ref, gate_b_ref,
          ew1_ref, eb1_ref, ew2_ref, eb2_ref, seg_ref, out_ref,
          ew1_scr, ew2b_scr):
    @pl.when(pl.program_id(0) == 0)
    def _():
        for e in range(N_EXPERTS):
            sl = pl.ds(e * D_FUSE_OUT, D_FUSE_OUT)
            ew1_scr[:, sl] = ew1_ref[e].astype(jnp.bfloat16)
            ew2b_scr[0:1, sl] = ew2_ref[e:e + 1].astype(jnp.bfloat16)
            ew2b_scr[1:2, sl] = eb1_ref[e:e + 1].astype(jnp.bfloat16)

    z1 = (jnp.dot(veh_ref[...], w1_ref[0:D_HID],
                  preferred_element_type=jnp.float32)
          + jnp.dot(ctx_ref[...], w1_ref[D_HID:2 * D_HID],
                    preferred_element_type=jnp.float32)
          + jnp.dot(route_ref[...], w1_ref[2 * D_HID:D_IN],
                    preferred_element_type=jnp.float32)
          + b1_ref[...])
    h = jax.nn.gelu(z1)
    mu = jnp.mean(h, axis=-1, keepdims=True)
    var = jnp.mean(h * h, axis=-1, keepdims=True) - mu * mu
    hn = (h - mu) / jnp.sqrt(var + 1e-5) * ln_g_ref[...] + ln_b_ref[...]
    f = jnp.dot(hn, w2_ref[...], preferred_element_type=jnp.float32) + b2_ref[...]

    lt = jax.lax.dot_general(
        gate_w_ref[...], f, (((0,), (1,)), ((), ())),
        preferred_element_type=jnp.float32) + gate_b_ref[...]
    rowi = jax.lax.broadcasted_iota(jnp.int32, lt.shape, 0)
    v1 = jnp.max(lt, axis=0, keepdims=True)
    i1 = jnp.min(jnp.where(lt == v1, rowi, N_EXPERTS), axis=0, keepdims=True)
    masked = jnp.where(rowi == i1, -jnp.inf, lt)
    v2 = jnp.max(masked, axis=0, keepdims=True)
    i2 = jnp.min(jnp.where(masked == v2, rowi, N_EXPERTS), axis=0, keepdims=True)
    g1 = 1.0 / (1.0 + jnp.exp(v2 - v1))
    g2 = 1.0 - g1
    wt = jnp.where(rowi == i1, g1, 0.0) + jnp.where(rowi == i2, g2, 0.0)

    pre = (jnp.dot(f.astype(jnp.bfloat16), ew1_scr[...],
                   preferred_element_type=jnp.float32).astype(jnp.bfloat16)
           + ew2b_scr[1:2])
    eh = _gelu_fast(pre)
    eyt = jax.lax.dot_general(
        seg_ref[...], eh * ew2b_scr[0:1], (((0,), (1,)), ((), ())),
        preferred_element_type=jnp.float32) + eb2_ref[...]

    out_ref[...] = jnp.sum(wt * eyt, axis=0, keepdims=True)[None]


@functools.partial(jax.jit, static_argnames=("interpret",))
def _run(veh_z, ctx, route_z, W1, b1, ln_g, ln_b, W2, b2, gate_W, gate_b,
         eW1, eb1, eW2, eb2, interpret=False):
    seg = jnp.repeat(jnp.eye(N_EXPERTS, dtype=jnp.bfloat16),
                     D_FUSE_OUT, axis=0)

    row = lambda i: (i, 0)
    fixed = lambda i: (0, 0)
    fixed3 = lambda i: (0, 0, 0)
    grid = NV // BLK
    out = pl.pallas_call(
        _body,
        grid=(grid,),
        in_specs=[
            pl.BlockSpec((BLK, D_HID), row),
            pl.BlockSpec((BLK, D_HID), row),
            pl.BlockSpec((BLK, D_ROUTE), row),
            pl.BlockSpec((D_IN, D_FUSE_HID), fixed),
            pl.BlockSpec((1, D_FUSE_HID), fixed),
            pl.BlockSpec((1, D_FUSE_HID), fixed),
            pl.BlockSpec((1, D_FUSE_HID), fixed),
            pl.BlockSpec((D_FUSE_HID, D_FUSE_OUT), fixed),
            pl.BlockSpec((1, D_FUSE_OUT), fixed),
            pl.BlockSpec((D_FUSE_OUT, N_EXPERTS), fixed),
            pl.BlockSpec((N_EXPERTS, 1), fixed),
            pl.BlockSpec((N_EXPERTS, D_FUSE_OUT, D_FUSE_OUT), fixed3),
            pl.BlockSpec((N_EXPERTS, D_FUSE_OUT), fixed),
            pl.BlockSpec((N_EXPERTS, D_FUSE_OUT), fixed),
            pl.BlockSpec((N_EXPERTS, 1), fixed),
            pl.BlockSpec((EH, N_EXPERTS), fixed),
        ],
        out_specs=pl.BlockSpec((1, 1, BLK), lambda i: (i, 0, 0)),
        out_shape=jax.ShapeDtypeStruct((grid, 1, BLK), jnp.float32),
        scratch_shapes=[
            pltpu.VMEM((D_FUSE_OUT, EH), jnp.bfloat16),
            pltpu.VMEM((2, EH), jnp.bfloat16),
        ],
        interpret=interpret,
    )(veh_z, ctx, route_z, W1, b1.reshape(1, -1),
      ln_g.reshape(1, -1), ln_b.reshape(1, -1), W2, b2.reshape(1, -1),
      gate_W, gate_b.reshape(N_EXPERTS, 1), eW1, eb1,
      eW2.reshape(N_EXPERTS, D_FUSE_OUT), eb2, seg)
    return out.reshape(NV)


def kernel(veh_z, ctx, route_z, W1, b1, ln_g, ln_b, W2, b2, gate_W, gate_b,
           eW1, eb1, eW2, eb2):
    return _run(veh_z, ctx, route_z, W1, b1, ln_g, ln_b, W2, b2, gate_W,
                gate_b, eW1, eb1, eW2, eb2)


# speedup vs baseline: 1.0076x; 1.0076x over previous
import functools

import jax
import jax.numpy as jnp
from jax.experimental import pallas as pl
from jax.experimental.pallas import tpu as pltpu

NV = 16384
D_HID = 128
D_ROUTE = 64
D_IN = 2 * D_HID + D_ROUTE
D_FUSE_HID = 256
D_FUSE_OUT = 192
N_EXPERTS = 8
EH = N_EXPERTS * D_FUSE_OUT
BLK = 2048


def _body(veh_ref, ctx_ref, route_ref, w1_ref, b1_ref,
          ln_g_ref, ln_b_ref, w2_ref, b2_ref, gate_w_ref, gate_b_ref,
          ew1_ref, eb1_ref, ew2_ref, eb2_ref, seg_ref, out_ref,
          ew1_scr, ew2b_scr):
    @pl.when(pl.program_id(0) == 0)
    def _():
        for e in range(N_EXPERTS):
            sl = pl.ds(e * D_FUSE_OUT, D_FUSE_OUT)
            ew1_scr[:, sl] = ew1_ref[e].astype(jnp.bfloat16)
            ew2b_scr[0:1, sl] = ew2_ref[e:e + 1].astype(jnp.bfloat16)
            ew2b_scr[1:2, sl] = eb1_ref[e:e + 1].astype(jnp.bfloat16)

    z1 = (jnp.dot(veh_ref[...], w1_ref[0:D_HID],
                  preferred_element_type=jnp.float32)
          + jnp.dot(ctx_ref[...], w1_ref[D_HID:2 * D_HID],
                    preferred_element_type=jnp.float32)
          + jnp.dot(route_ref[...], w1_ref[2 * D_HID:D_IN],
                    preferred_element_type=jnp.float32)
          + b1_ref[...])
    h = jax.nn.gelu(z1)
    mu = jnp.mean(h, axis=-1, keepdims=True)
    var = jnp.mean(h * h, axis=-1, keepdims=True) - mu * mu
    hn = (h - mu) / jnp.sqrt(var + 1e-5) * ln_g_ref[...] + ln_b_ref[...]
    f = jnp.dot(hn, w2_ref[...], preferred_element_type=jnp.float32) + b2_ref[...]

    lt = jax.lax.dot_general(
        gate_w_ref[...], f, (((0,), (1,)), ((), ())),
        preferred_element_type=jnp.float32) + gate_b_ref[...]
    rowi = jax.lax.broadcasted_iota(jnp.int32, lt.shape, 0)
    v1 = jnp.max(lt, axis=0, keepdims=True)
    i1 = jnp.min(jnp.where(lt == v1, rowi, N_EXPERTS), axis=0, keepdims=True)
    masked = jnp.where(rowi == i1, -jnp.inf, lt)
    v2 = jnp.max(masked, axis=0, keepdims=True)
    i2 = jnp.min(jnp.where(masked == v2, rowi, N_EXPERTS), axis=0, keepdims=True)
    g1 = 1.0 / (1.0 + jnp.exp(v2 - v1))
    g2 = 1.0 - g1
    wt = jnp.where(rowi == i1, g1, 0.0) + jnp.where(rowi == i2, g2, 0.0)

    pre = (jnp.dot(f.astype(jnp.bfloat16), ew1_scr[...],
                   preferred_element_type=jnp.float32).astype(jnp.bfloat16)
           + ew2b_scr[1:2])
    eh = jax.nn.gelu(pre)
    eyt = jax.lax.dot_general(
        seg_ref[...], eh * ew2b_scr[0:1], (((0,), (1,)), ((), ())),
        preferred_element_type=jnp.float32) + eb2_ref[...]

    out_ref[...] = jnp.sum(wt * eyt, axis=0, keepdims=True)[None]


@functools.partial(jax.jit, static_argnames=("interpret",))
def _run(veh_z, ctx, route_z, W1, b1, ln_g, ln_b, W2, b2, gate_W, gate_b,
         eW1, eb1, eW2, eb2, interpret=False):
    seg = jnp.repeat(jnp.eye(N_EXPERTS, dtype=jnp.bfloat16),
                     D_FUSE_OUT, axis=0)

    row = lambda i: (i, 0)
    fixed = lambda i: (0, 0)
    fixed3 = lambda i: (0, 0, 0)
    grid = NV // BLK
    out = pl.pallas_call(
        _body,
        grid=(grid,),
        in_specs=[
            pl.BlockSpec((BLK, D_HID), row),
            pl.BlockSpec((BLK, D_HID), row),
            pl.BlockSpec((BLK, D_ROUTE), row),
            pl.BlockSpec((D_IN, D_FUSE_HID), fixed),
            pl.BlockSpec((1, D_FUSE_HID), fixed),
            pl.BlockSpec((1, D_FUSE_HID), fixed),
            pl.BlockSpec((1, D_FUSE_HID), fixed),
            pl.BlockSpec((D_FUSE_HID, D_FUSE_OUT), fixed),
            pl.BlockSpec((1, D_FUSE_OUT), fixed),
            pl.BlockSpec((D_FUSE_OUT, N_EXPERTS), fixed),
            pl.BlockSpec((N_EXPERTS, 1), fixed),
            pl.BlockSpec((N_EXPERTS, D_FUSE_OUT, D_FUSE_OUT), fixed3),
            pl.BlockSpec((N_EXPERTS, D_FUSE_OUT), fixed),
            pl.BlockSpec((N_EXPERTS, D_FUSE_OUT), fixed),
            pl.BlockSpec((N_EXPERTS, 1), fixed),
            pl.BlockSpec((EH, N_EXPERTS), fixed),
        ],
        out_specs=pl.BlockSpec((1, 1, BLK), lambda i: (i, 0, 0)),
        out_shape=jax.ShapeDtypeStruct((grid, 1, BLK), jnp.float32),
        scratch_shapes=[
            pltpu.VMEM((D_FUSE_OUT, EH), jnp.bfloat16),
            pltpu.VMEM((2, EH), jnp.bfloat16),
        ],
        interpret=interpret,
    )(veh_z, ctx, route_z, W1, b1.reshape(1, -1),
      ln_g.reshape(1, -1), ln_b.reshape(1, -1), W2, b2.reshape(1, -1),
      gate_W, gate_b.reshape(N_EXPERTS, 1), eW1, eb1,
      eW2.reshape(N_EXPERTS, D_FUSE_OUT), eb2, seg)
    return out.reshape(NV)


def kernel(veh_z, ctx, route_z, W1, b1, ln_g, ln_b, W2, b2, gate_W, gate_b,
           eW1, eb1, eW2, eb2):
    return _run(veh_z, ctx, route_z, W1, b1, ln_g, ln_b, W2, b2, gate_W,
                gate_b, eW1, eb1, eW2, eb2)
